# baseline (speedup 1.0000x reference)
import functools

import jax
import jax.numpy as jnp
import numpy as np
from jax.experimental import pallas as pl
from jax.experimental.pallas import tpu as pltpu

T, H = 8, 8
NB = 2 ** H
D_IN, D_OUT = 1024, 4096
S = 2048
BO = 512
CS = 512


def _fused_body(x_ref, w_ref, b_ref, p_ref, pk_ref, pkt_ref, o_ref,
                xbf_ref, xoh_ref, woh_ref):
    f32 = jnp.float32
    bf16 = jnp.bfloat16
    hi = jax.lax.Precision.HIGHEST
    pbf = p_ref[...].astype(bf16)

    @pl.when(pl.program_id(0) == 0)
    def _():
        for c in range(S // CS):
            xc = x_ref[pl.ds(c * CS, CS), :].astype(bf16)
            xbf_ref[pl.ds(c * CS, CS), :] = xc
            s = jax.lax.dot_general(xc, pbf, (((1,), (1,)), ((), ())),
                                    preferred_element_type=f32)
            bits = (s > 0.0).astype(f32)
            codes = jax.lax.dot_general(bits, pk_ref[...],
                                        (((1,), (0,)), ((), ())),
                                        precision=hi,
                                        preferred_element_type=f32)
            iota = jax.lax.broadcasted_iota(jnp.int32, (CS, NB), 1).astype(f32)
            for t in range(T):
                oh = (codes[:, t:t + 1] == iota).astype(bf16)
                xoh_ref[pl.ds(c * CS, CS), pl.ds(t * NB, NB)] = oh

    wbf = w_ref[...].astype(bf16)
    sw = jax.lax.dot_general(wbf, pbf, (((1,), (1,)), ((), ())),
                             preferred_element_type=f32)
    wbits = (sw > 0.0).astype(f32)
    wct = jax.lax.dot_general(pkt_ref[...], wbits, (((1,), (1,)), ((), ())),
                              precision=hi,
                              preferred_element_type=f32)
    iota_w = jax.lax.broadcasted_iota(jnp.int32, (NB, BO), 0).astype(f32)
    for t in range(T):
        woh_ref[pl.ds(t * NB, NB), :] = (iota_w == wct[t:t + 1, :]).astype(bf16)

    bias = b_ref[...]
    woh = woh_ref[...]

    for c in range(S // CS):
        xc = xbf_ref[pl.ds(c * CS, CS), :]
        d = jax.lax.dot_general(xc, wbf, (((1,), (1,)), ((), ())),
                                preferred_element_type=f32) + bias
        cnt = jax.lax.dot_general(xoh_ref[pl.ds(c * CS, CS), :], woh,
                                  (((1,), (0,)), ((), ())),
                                  preferred_element_type=f32)
        o_ref[pl.ds(c * CS, CS), :] = jnp.where(cnt > 0.0, d, 0.0)


@functools.partial(jax.jit, static_argnames=())
def kernel(x, W, b, proj):
    B = x.shape[0]
    x2 = x.reshape(B * S, D_IN)
    proj2 = proj.reshape(T * H, D_IN)
    b2 = b.reshape(1, D_OUT)
    pk = np.zeros((T * H, T), dtype=np.float32)
    for t in range(T):
        for h in range(H):
            pk[t * H + h, t] = float(2 ** h)
    pkt = jnp.asarray(pk.T.copy())
    pk = jnp.asarray(pk)

    out = pl.pallas_call(
        _fused_body,
        grid=(D_OUT // BO,),
        in_specs=[
            pl.BlockSpec((B * S, D_IN), lambda o: (0, 0)),
            pl.BlockSpec((BO, D_IN), lambda o: (o, 0)),
            pl.BlockSpec((1, BO), lambda o: (0, o)),
            pl.BlockSpec((T * H, D_IN), lambda o: (0, 0)),
            pl.BlockSpec((T * H, T), lambda o: (0, 0)),
            pl.BlockSpec((T, T * H), lambda o: (0, 0)),
        ],
        out_specs=pl.BlockSpec((B * S, BO), lambda o: (0, o)),
        out_shape=jax.ShapeDtypeStruct((B * S, D_OUT), jnp.float32),
        scratch_shapes=[pltpu.VMEM((B * S, D_IN), jnp.bfloat16),
                        pltpu.VMEM((B * S, T * NB), jnp.bfloat16),
                        pltpu.VMEM((T * NB, BO), jnp.bfloat16)],
    )(x2, W, b2, proj2, pk, pkt)
    return out.reshape(B, S, D_OUT)


# speedup vs baseline: 1.4452x; 1.4452x over previous
import functools

import jax
import jax.numpy as jnp
import numpy as np
from jax.experimental import pallas as pl
from jax.experimental.pallas import tpu as pltpu

T, H = 8, 8
D_IN, D_OUT = 1024, 4096
S = 2048
BO = 512
CS = 512


def _prep_body(x_ref, p_ref, pk_ref, xbf_ref, xc_ref):
    f32 = jnp.float32
    bf16 = jnp.bfloat16
    hi = jax.lax.Precision.HIGHEST
    pbf = p_ref[...].astype(bf16)
    for c in range(S // CS):
        xc = x_ref[pl.ds(c * CS, CS), :].astype(bf16)
        xbf_ref[pl.ds(c * CS, CS), :] = xc
        s = jax.lax.dot_general(xc, pbf, (((1,), (1,)), ((), ())),
                                preferred_element_type=f32)
        bits = (s > 0.0).astype(f32)
        codes = jax.lax.dot_general(bits, pk_ref[...],
                                    (((1,), (0,)), ((), ())),
                                    precision=hi,
                                    preferred_element_type=f32)
        xc_ref[pl.ds(c * CS, CS), :] = codes.astype(bf16)


def _main_body(xbf_ref, xc_ref, w_ref, b_ref, p_ref, pkt_ref, o_ref):
    f32 = jnp.float32
    bf16 = jnp.bfloat16
    hi = jax.lax.Precision.HIGHEST
    pbf = p_ref[...].astype(bf16)

    wbf = w_ref[...].astype(bf16)
    sw = jax.lax.dot_general(wbf, pbf, (((1,), (1,)), ((), ())),
                             preferred_element_type=f32)
    wbits = (sw > 0.0).astype(f32)
    wct = jax.lax.dot_general(pkt_ref[...], wbits, (((1,), (1,)), ((), ())),
                              precision=hi,
                              preferred_element_type=f32).astype(bf16)

    bias = b_ref[...]
    for c in range(S // CS):
        xc = xbf_ref[pl.ds(c * CS, CS), :]
        d = jax.lax.dot_general(xc, wbf, (((1,), (1,)), ((), ())),
                                preferred_element_type=f32) + bias
        codes = xc_ref[pl.ds(c * CS, CS), :]
        ms = [codes[:, t:t + 1] == wct[t:t + 1, :] for t in range(T)]
        while len(ms) > 1:
            ms = [jnp.logical_or(a, b) for a, b in zip(ms[::2], ms[1::2])]
        o_ref[pl.ds(c * CS, CS), :] = jnp.where(ms[0], d, 0.0)


@functools.partial(jax.jit, static_argnames=())
def kernel(x, W, b, proj):
    B = x.shape[0]
    x2 = x.reshape(B * S, D_IN)
    proj2 = proj.reshape(T * H, D_IN)
    b2 = b.reshape(1, D_OUT)
    pk = np.zeros((T * H, T), dtype=np.float32)
    for t in range(T):
        for h in range(H):
            pk[t * H + h, t] = float(2 ** h)
    pkt = jnp.asarray(pk.T.copy())
    pk = jnp.asarray(pk)

    xbf, xcodes = pl.pallas_call(
        _prep_body,
        grid=(1,),
        in_specs=[
            pl.BlockSpec((B * S, D_IN), lambda o: (0, 0)),
            pl.BlockSpec((T * H, D_IN), lambda o: (0, 0)),
            pl.BlockSpec((T * H, T), lambda o: (0, 0)),
        ],
        out_specs=[pl.BlockSpec((B * S, D_IN), lambda o: (0, 0)),
                   pl.BlockSpec((B * S, T), lambda o: (0, 0))],
        out_shape=[jax.ShapeDtypeStruct((B * S, D_IN), jnp.bfloat16),
                   jax.ShapeDtypeStruct((B * S, T), jnp.bfloat16)],
    )(x2, proj2, pk)

    out = pl.pallas_call(
        _main_body,
        grid=(D_OUT // BO,),
        in_specs=[
            pl.BlockSpec((B * S, D_IN), lambda o: (0, 0)),
            pl.BlockSpec((B * S, T), lambda o: (0, 0)),
            pl.BlockSpec((BO, D_IN), lambda o: (o, 0)),
            pl.BlockSpec((1, BO), lambda o: (0, o)),
            pl.BlockSpec((T * H, D_IN), lambda o: (0, 0)),
            pl.BlockSpec((T, T * H), lambda o: (0, 0)),
        ],
        out_specs=pl.BlockSpec((B * S, BO), lambda o: (0, o)),
        out_shape=jax.ShapeDtypeStruct((B * S, D_OUT), jnp.float32),
        compiler_params=pltpu.CompilerParams(
            dimension_semantics=("parallel",)),
    )(xbf, xcodes, W, b2, proj2, pkt)
    return out.reshape(B, S, D_OUT)


# speedup vs baseline: 1.4700x; 1.0172x over previous
import functools

import jax
import jax.numpy as jnp
import numpy as np
from jax.experimental import pallas as pl
from jax.experimental.pallas import tpu as pltpu

T, H = 8, 8
D_IN, D_OUT = 1024, 4096
S = 2048
BO = 1024
CS = 256


def _fused_body(x_ref, w_ref, b_ref, p_ref, pk_ref, pkt_ref, o_ref,
                xbf_ref, xc_ref):
    f32 = jnp.float32
    bf16 = jnp.bfloat16
    hi = jax.lax.Precision.HIGHEST
    pbf = p_ref[...].astype(bf16)

    @pl.when(pl.program_id(0) == 0)
    def _():
        for c in range(S // CS):
            xc = x_ref[pl.ds(c * CS, CS), :].astype(bf16)
            xbf_ref[pl.ds(c * CS, CS), :] = xc
            s = jax.lax.dot_general(xc, pbf, (((1,), (1,)), ((), ())),
                                    preferred_element_type=f32)
            bits = (s > 0.0).astype(f32)
            codes = jax.lax.dot_general(bits, pk_ref[...],
                                        (((1,), (0,)), ((), ())),
                                        precision=hi,
                                        preferred_element_type=f32)
            xc_ref[pl.ds(c * CS, CS), :] = codes.astype(bf16)

    wbf = w_ref[...].astype(bf16)
    sw = jax.lax.dot_general(wbf, pbf, (((1,), (1,)), ((), ())),
                             preferred_element_type=f32)
    wbits = (sw > 0.0).astype(f32)
    wct = jax.lax.dot_general(pkt_ref[...], wbits, (((1,), (1,)), ((), ())),
                              precision=hi,
                              preferred_element_type=f32).astype(bf16)

    bias = b_ref[...]
    nc = S // CS

    def _dense(c):
        xc = xbf_ref[pl.ds(c * CS, CS), :]
        d = jax.lax.dot_general(xc, wbf, (((1,), (1,)), ((), ())),
                                preferred_element_type=f32)
        return d + bias

    dense_p = _dense(0)
    for c in range(nc):
        dense_n = _dense(c + 1) if c + 1 < nc else None
        codes = xc_ref[pl.ds(c * CS, CS), :]
        ms = [codes[:, t:t + 1] == wct[t:t + 1, :] for t in range(T)]
        while len(ms) > 1:
            ms = [jnp.logical_or(a, b) for a, b in zip(ms[::2], ms[1::2])]
        o_ref[pl.ds(c * CS, CS), :] = jnp.where(ms[0], dense_p, 0.0)
        dense_p = dense_n


@functools.partial(jax.jit, static_argnames=())
def kernel(x, W, b, proj):
    B = x.shape[0]
    x2 = x.reshape(B * S, D_IN)
    proj2 = proj.reshape(T * H, D_IN)
    b2 = b.reshape(1, D_OUT)
    pk = np.zeros((T * H, T), dtype=np.float32)
    for t in range(T):
        for h in range(H):
            pk[t * H + h, t] = float(2 ** h)
    pkt = jnp.asarray(pk.T.copy())
    pk = jnp.asarray(pk)

    out = pl.pallas_call(
        _fused_body,
        grid=(D_OUT // BO,),
        in_specs=[
            pl.BlockSpec((B * S, D_IN), lambda o: (0, 0)),
            pl.BlockSpec((BO, D_IN), lambda o: (o, 0)),
            pl.BlockSpec((1, BO), lambda o: (0, o)),
            pl.BlockSpec((T * H, D_IN), lambda o: (0, 0)),
            pl.BlockSpec((T * H, T), lambda o: (0, 0)),
            pl.BlockSpec((T, T * H), lambda o: (0, 0)),
        ],
        out_specs=pl.BlockSpec((B * S, BO), lambda o: (0, o)),
        out_shape=jax.ShapeDtypeStruct((B * S, D_OUT), jnp.float32),
        scratch_shapes=[pltpu.VMEM((B * S, D_IN), jnp.bfloat16),
                        pltpu.VMEM((B * S, T), jnp.bfloat16)],
    )(x2, W, b2, proj2, pk, pkt)
    return out.reshape(B, S, D_OUT)


# speedup vs baseline: 1.5529x; 1.0564x over previous
import functools

import jax
import jax.numpy as jnp
import numpy as np
from jax.experimental import pallas as pl
from jax.experimental.pallas import tpu as pltpu

T, H = 8, 8
D_IN, D_OUT = 1024, 4096
S = 2048
BO = 512
CS = 512


def _fused_body(x_ref, w_ref, b_ref, p_ref, pk_ref, pkt_ref, o_ref,
                xbf_ref, xc_ref):
    f32 = jnp.float32
    bf16 = jnp.bfloat16
    hi = jax.lax.Precision.HIGHEST
    pbf = p_ref[...].astype(bf16)

    @pl.when(pl.program_id(0) == 0)
    def _():
        for c in range(S // CS):
            xc = x_ref[pl.ds(c * CS, CS), :].astype(bf16)
            xbf_ref[pl.ds(c * CS, CS), :] = xc
            s = jax.lax.dot_general(xc, pbf, (((1,), (1,)), ((), ())),
                                    preferred_element_type=f32)
            bits = (s > 0.0).astype(f32)
            codes = jax.lax.dot_general(bits, pk_ref[...],
                                        (((1,), (0,)), ((), ())),
                                        precision=hi,
                                        preferred_element_type=f32)
            xc_ref[pl.ds(c * CS, CS), :] = codes.astype(bf16)

    wbf = w_ref[...].astype(bf16)
    sw = jax.lax.dot_general(wbf, pbf, (((1,), (1,)), ((), ())),
                             preferred_element_type=f32)
    wbits = (sw > 0.0).astype(f32)
    wct = jax.lax.dot_general(pkt_ref[...], wbits, (((1,), (1,)), ((), ())),
                              precision=hi,
                              preferred_element_type=f32).astype(bf16)

    bias = b_ref[...]
    nc = S // CS

    def _dense(c):
        xc = xbf_ref[pl.ds(c * CS, CS), :]
        d = jax.lax.dot_general(xc, wbf, (((1,), (1,)), ((), ())),
                                preferred_element_type=f32)
        return d + bias

    dense_p = _dense(0)
    for c in range(nc):
        dense_n = _dense(c + 1) if c + 1 < nc else None
        codes = xc_ref[pl.ds(c * CS, CS), :]
        ms = [codes[:, t:t + 1] == wct[t:t + 1, :] for t in range(T)]
        while len(ms) > 1:
            ms = [jnp.logical_or(a, b) for a, b in zip(ms[::2], ms[1::2])]
        o_ref[pl.ds(c * CS, CS), :] = jnp.where(ms[0], dense_p, 0.0)
        dense_p = dense_n


@functools.partial(jax.jit, static_argnames=())
def kernel(x, W, b, proj):
    B = x.shape[0]
    x2 = x.reshape(B * S, D_IN)
    proj2 = proj.reshape(T * H, D_IN)
    b2 = b.reshape(1, D_OUT)
    pk = np.zeros((T * H, T), dtype=np.float32)
    for t in range(T):
        for h in range(H):
            pk[t * H + h, t] = float(2 ** h)
    pkt = jnp.asarray(pk.T.copy())
    pk = jnp.asarray(pk)

    out = pl.pallas_call(
        _fused_body,
        grid=(D_OUT // BO,),
        in_specs=[
            pl.BlockSpec((B * S, D_IN), lambda o: (0, 0)),
            pl.BlockSpec((BO, D_IN), lambda o: (o, 0)),
            pl.BlockSpec((1, BO), lambda o: (0, o)),
            pl.BlockSpec((T * H, D_IN), lambda o: (0, 0)),
            pl.BlockSpec((T * H, T), lambda o: (0, 0)),
            pl.BlockSpec((T, T * H), lambda o: (0, 0)),
        ],
        out_specs=pl.BlockSpec((B * S, BO), lambda o: (0, o)),
        out_shape=jax.ShapeDtypeStruct((B * S, D_OUT), jnp.float32),
        scratch_shapes=[pltpu.VMEM((B * S, D_IN), jnp.bfloat16),
                        pltpu.VMEM((B * S, T), jnp.bfloat16)],
    )(x2, W, b2, proj2, pk, pkt)
    return out.reshape(B, S, D_OUT)
